# baseline (speedup 1.0000x reference)
import jax
import jax.numpy as jnp
from jax.experimental import pallas as pl


def _gnn_kernel(unt_ref, adj_ref, mask_ref, sel1_ref, sel2_ref,
                wemb_ref, bemb_ref,
                wroot0_ref, wrelc0_ref, b0_ref,
                wroot1_ref, wrelc1_ref, b1_ref,
                wpol_ref, bpol_ref, wbase_ref, bbase_ref,
                logits_ref, base_ref, act_ref):
    f32 = jnp.float32
    A = wpol_ref.shape[1]

    adjf = adj_ref[0].astype(f32)
    deg = jnp.sum(adjf, axis=0, keepdims=True)
    adjh = adjf * (1.0 / jnp.maximum(deg, 1.0))

    xT = jax.lax.dot_general(wemb_ref[...], unt_ref[0],
                             (((0,), (0,)), ((), ())),
                             preferred_element_type=f32) + bemb_ref[...]

    mask = mask_ref[...]
    sel1 = sel1_ref[...]
    sel2 = sel2_ref[...]

    def rgcn(xT, wroot, wrelc, bias):
        hall = jax.lax.dot_general(wrelc, xT, (((0,), (0,)), ((), ())),
                                   preferred_element_type=f32)
        g = jax.lax.dot_general(hall, adjh, (((1,), (0,)), ((), ())),
                                precision=jax.lax.Precision.HIGHEST,
                                preferred_element_type=f32)
        s1g = jax.lax.dot_general(sel1, g * mask, (((1,), (0,)), ((), ())),
                                  precision=jax.lax.Precision.HIGHEST,
                                  preferred_element_type=f32)
        aggT = jax.lax.dot_general(s1g, sel2, (((1,), (0,)), ((), ())),
                                   precision=jax.lax.Precision.HIGHEST,
                                   preferred_element_type=f32)
        rootT = jax.lax.dot_general(wroot, xT, (((0,), (0,)), ((), ())),
                                    preferred_element_type=f32)
        return jax.nn.relu(rootT + bias + aggT)

    xT = rgcn(xT, wroot0_ref[...], wrelc0_ref[...], b0_ref[...])
    xT = rgcn(xT, wroot1_ref[...], wrelc1_ref[...], b1_ref[...])

    pooled = jnp.max(xT, axis=1, keepdims=True)
    logits = jax.lax.dot_general(pooled, wpol_ref[...], (((0,), (0,)), ((), ())),
                                 preferred_element_type=f32) + bpol_ref[...]
    base = jax.lax.dot_general(pooled, wbase_ref[...], (((0,), (0,)), ((), ())),
                               preferred_element_type=f32) + bbase_ref[...]

    logits_ref[0] = logits
    base_ref[0] = base
    m = jnp.max(logits, axis=1, keepdims=True)
    iota = jax.lax.broadcasted_iota(jnp.int32, logits.shape, 1)
    act_ref[0] = jnp.min(jnp.where(logits == m, iota, A), axis=1, keepdims=True)


def kernel(unary_tensor, binary_tensor, W_emb, b_emb, Wroot0, Wrel0, b0,
           Wroot1, Wrel1, b1, W_pol, b_pol, W_base, b_base):
    Tt, Bb, N, F = unary_tensor.shape
    R = binary_tensor.shape[-1]
    D = W_emb.shape[1]
    A = W_pol.shape[1]
    BT = Tt * Bb
    NR = N * R
    RD = R * D
    f32 = jnp.float32

    unt = unary_tensor.reshape(BT, N, F).astype(f32).transpose(0, 2, 1)
    adj = binary_tensor.reshape(BT, N, NR)

    cc = jnp.arange(NR, dtype=jnp.int32)
    mask = (jnp.arange(RD, dtype=jnp.int32)[:, None] // D == cc[None, :] % R).astype(f32)
    sel1 = (jnp.arange(D, dtype=jnp.int32)[:, None] == jnp.arange(RD, dtype=jnp.int32)[None, :] % D).astype(f32)
    sel2 = (cc[:, None] // R == jnp.arange(N, dtype=jnp.int32)[None, :]).astype(f32)
    wrelc0 = Wrel0.transpose(1, 0, 2).reshape(D, RD)
    wrelc1 = Wrel1.transpose(1, 0, 2).reshape(D, RD)

    full = lambda *shape: pl.BlockSpec(shape, lambda b: (0,) * len(shape))
    in_specs = [
        pl.BlockSpec((1, F, N), lambda b: (b, 0, 0)),
        pl.BlockSpec((1, N, NR), lambda b: (b, 0, 0)),
        full(RD, NR), full(D, RD), full(NR, N),
        full(F, D), full(D, 1),
        full(D, D), full(D, RD), full(D, 1),
        full(D, D), full(D, RD), full(D, 1),
        full(D, A), full(1, A), full(D, 1), full(1, 1),
    ]
    out_specs = [
        pl.BlockSpec((1, 1, A), lambda b: (b, 0, 0)),
        pl.BlockSpec((1, 1, 1), lambda b: (b, 0, 0)),
        pl.BlockSpec((1, 1, 1), lambda b: (b, 0, 0)),
    ]
    logits, base, act = pl.pallas_call(
        _gnn_kernel,
        grid=(BT,),
        in_specs=in_specs,
        out_specs=out_specs,
        out_shape=[
            jax.ShapeDtypeStruct((BT, 1, A), f32),
            jax.ShapeDtypeStruct((BT, 1, 1), f32),
            jax.ShapeDtypeStruct((BT, 1, 1), jnp.int32),
        ],
    )(unt, adj, mask, sel1, sel2,
      W_emb, b_emb.reshape(D, 1),
      Wroot0, wrelc0, b0.reshape(D, 1),
      Wroot1, wrelc1, b1.reshape(D, 1),
      W_pol, b_pol.reshape(1, A), W_base, b_base.reshape(1, 1))

    return (logits.reshape(Tt, Bb, A),
            base.reshape(Tt, Bb),
            act.reshape(Tt, Bb))


# speedup vs baseline: 1.1887x; 1.1887x over previous
import numpy as np
import jax
import jax.numpy as jnp
from jax.experimental import pallas as pl


def _gnn_kernel(unt_ref, adj_ref, mask_ref, sel2_ref,
                wemb_ref, bemb_ref,
                wroot0_ref, wrelc0_ref, b0_ref,
                wroot1_ref, wrelc1_ref, b1_ref,
                wpol_ref, bpol_ref, wbase_ref, bbase_ref,
                logits_ref, base_ref, act_ref):
    f32 = jnp.float32
    A = wpol_ref.shape[1]
    D = wroot0_ref.shape[0]
    R = mask_ref.shape[0] // D

    adjf = adj_ref[0].astype(f32)
    deg = jnp.sum(adjf, axis=0, keepdims=True)
    adjh = adjf * (1.0 / jnp.maximum(deg, 1.0))

    xT = jax.lax.dot_general(wemb_ref[...], unt_ref[0],
                             (((0,), (0,)), ((), ())),
                             preferred_element_type=f32) + bemb_ref[...]

    mask = mask_ref[...]
    sel2 = sel2_ref[...]

    def rgcn(xT, wroot, wrelc, bias):
        hall = jax.lax.dot_general(wrelc, xT, (((0,), (0,)), ((), ())),
                                   preferred_element_type=f32)
        g = jax.lax.dot_general(hall, adjh, (((1,), (0,)), ((), ())),
                                preferred_element_type=f32)
        gm = g * mask
        s1g = gm[0 * D:1 * D] + gm[1 * D:2 * D] + gm[2 * D:3 * D] + gm[3 * D:4 * D]
        aggT = jax.lax.dot_general(s1g, sel2, (((1,), (0,)), ((), ())),
                                   precision=jax.lax.Precision.HIGHEST,
                                   preferred_element_type=f32)
        rootT = jax.lax.dot_general(wroot, xT, (((0,), (0,)), ((), ())),
                                    preferred_element_type=f32)
        return jax.nn.relu(rootT + bias + aggT)

    xT = rgcn(xT, wroot0_ref[...], wrelc0_ref[...], b0_ref[...])
    xT = rgcn(xT, wroot1_ref[...], wrelc1_ref[...], b1_ref[...])

    pooled = jnp.max(xT, axis=1, keepdims=True)
    logits = jax.lax.dot_general(pooled, wpol_ref[...], (((0,), (0,)), ((), ())),
                                 preferred_element_type=f32) + bpol_ref[...]
    base = jax.lax.dot_general(pooled, wbase_ref[...], (((0,), (0,)), ((), ())),
                               preferred_element_type=f32) + bbase_ref[...]

    logits_ref[0] = logits
    base_ref[0] = base
    m = jnp.max(logits, axis=1, keepdims=True)
    iota = jax.lax.broadcasted_iota(jnp.int32, logits.shape, 1)
    act_ref[0] = jnp.min(jnp.where(logits == m, iota, A), axis=1, keepdims=True)


def kernel(unary_tensor, binary_tensor, W_emb, b_emb, Wroot0, Wrel0, b0,
           Wroot1, Wrel1, b1, W_pol, b_pol, W_base, b_base):
    Tt, Bb, N, F = unary_tensor.shape
    R = binary_tensor.shape[-1]
    D = W_emb.shape[1]
    A = W_pol.shape[1]
    BT = Tt * Bb
    NR = N * R
    RD = R * D
    f32 = jnp.float32

    unt = unary_tensor.reshape(BT, N, F).astype(f32).transpose(0, 2, 1)
    adj = binary_tensor.reshape(BT, N, NR)

    cc = np.arange(NR, dtype=np.int64)
    mask = jnp.asarray((np.arange(RD)[:, None] // D == cc[None, :] % R), dtype=f32)
    sel2 = jnp.asarray((cc[:, None] // R == np.arange(N)[None, :]), dtype=f32)
    wrelc0 = Wrel0.transpose(1, 0, 2).reshape(D, RD)
    wrelc1 = Wrel1.transpose(1, 0, 2).reshape(D, RD)

    full = lambda *shape: pl.BlockSpec(shape, lambda b: (0,) * len(shape))
    in_specs = [
        pl.BlockSpec((1, F, N), lambda b: (b, 0, 0)),
        pl.BlockSpec((1, N, NR), lambda b: (b, 0, 0)),
        full(RD, NR), full(NR, N),
        full(F, D), full(D, 1),
        full(D, D), full(D, RD), full(D, 1),
        full(D, D), full(D, RD), full(D, 1),
        full(D, A), full(1, A), full(D, 1), full(1, 1),
    ]
    out_specs = [
        pl.BlockSpec((1, 1, A), lambda b: (b, 0, 0)),
        pl.BlockSpec((1, 1, 1), lambda b: (b, 0, 0)),
        pl.BlockSpec((1, 1, 1), lambda b: (b, 0, 0)),
    ]
    logits, base, act = pl.pallas_call(
        _gnn_kernel,
        grid=(BT,),
        in_specs=in_specs,
        out_specs=out_specs,
        out_shape=[
            jax.ShapeDtypeStruct((BT, 1, A), f32),
            jax.ShapeDtypeStruct((BT, 1, 1), f32),
            jax.ShapeDtypeStruct((BT, 1, 1), jnp.int32),
        ],
    )(unt, adj, mask, sel2,
      W_emb, b_emb.reshape(D, 1),
      Wroot0, wrelc0, b0.reshape(D, 1),
      Wroot1, wrelc1, b1.reshape(D, 1),
      W_pol, b_pol.reshape(1, A), W_base, b_base.reshape(1, 1))

    return (logits.reshape(Tt, Bb, A),
            base.reshape(Tt, Bb),
            act.reshape(Tt, Bb))
